# baseline (speedup 1.0000x reference)
import functools

import jax
import jax.numpy as jnp
from jax import lax
from jax.experimental import pallas as pl
from jax.experimental.pallas import tpu as pltpu
from jax.experimental.pallas import tpu_sc as plsc

NC = 2
NS = 16
L = 16
K = 128


def _make_sc_spmm(n, half, ch):
    npt = n // NS
    n_chunks = npt // K
    mesh = plsc.VectorSubcoreMesh(
        core_axis_name="c", subcore_axis_name="s", num_cores=NC,
        num_subcores=NS)

    @functools.partial(
        pl.kernel,
        out_type=[
            jax.ShapeDtypeStruct((NC, n, half), jnp.float32),
            jax.ShapeDtypeStruct((NC, n, half), jnp.float32),
        ],
        mesh=mesh,
        scratch_types=[
            pltpu.VMEM((4, 3, K), jnp.int32),
            pltpu.VMEM((4, K, 2 * half), jnp.float32),
            pltpu.VMEM((2, K, half), jnp.float32),
            pltpu.VMEM_SHARED((n, half), jnp.float32),
            pltpu.SemaphoreType.DMA,
            pltpu.SemaphoreType.DMA,
            pltpu.SemaphoreType.DMA,
            pltpu.SemaphoreType.DMA,
            pltpu.SemaphoreType.DMA,
            pltpu.SemaphoreType.DMA,
        ],
        compiler_params=pltpu.CompilerParams(use_tc_tiling_on_sc=False,
                                             needs_layout_passes=False),
    )
    def sc_kernel(table_h, ed_h, o1_h, o2_h,
                  eb, rows_v, m_v, acc,
                  esem0, esem1, gsem0, gsem1, ssem0, ssem1):
        cid = lax.axis_index("c")
        sid = lax.axis_index("s")
        esems = (esem0, esem1)
        gsems = (gsem0, gsem1)
        ssems = (ssem0, ssem1)

        base = sid * npt
        zero = jnp.zeros((L,), jnp.float32)

        def zero_acc():
            def zb(k, carry):
                for j in range(half // L):
                    m_v[0, k, pl.ds(j * L, L)] = zero
                return carry

            lax.fori_loop(0, K, zb, 0)
            for i in range(n_chunks):
                pltpu.sync_copy(m_v.at[0], acc.at[pl.ds(base + i * K, K)])

        def spmm_pass(square, o_h):
            zero_acc()
            plsc.subcore_barrier()

            def compute(b, q):
                def group(g, carry2):
                    vv = plsc.bitcast(eb[q, 2, pl.ds(g * L, L)], jnp.float32)
                    for k in range(L):
                        v = vv[k]
                        kk = g * L + k
                        for j in range(half // L):
                            r = rows_v[b, kk, pl.ds(j * L, L)]
                            m = r * v
                            if square:
                                m = m * r
                            m_v[b, kk, pl.ds(j * L, L)] = m
                    return carry2

                lax.fori_loop(0, K // L, group, 0)

            pltpu.async_copy(ed_h.at[cid, sid, 0], eb.at[0], esem0)
            pltpu.async_copy(ed_h.at[cid, sid, 1], eb.at[1], esem1)
            pltpu.make_async_copy(ed_h.at[cid, sid, 0], eb.at[0],
                                  esem0).wait()
            pltpu.async_copy(table_h.at[eb.at[0, 0]], rows_v.at[0], gsem0)

            def quad(p, carry):
                for qb in range(4):
                    c = 4 * p + qb
                    b = qb % 2
                    if qb < 2:
                        @pl.when(p > 0)
                        def _():
                            pltpu.make_async_copy(table_h.at[eb.at[qb, 0]],
                                                  rows_v.at[(qb + 2) % 4],
                                                  gsem0).wait()
                    else:
                        pltpu.make_async_copy(table_h.at[eb.at[qb, 0]],
                                              rows_v.at[(qb + 2) % 4],
                                              gsem0).wait()
                    if qb < 2:
                        pltpu.async_copy(ed_h.at[cid, sid, c + 2],
                                         eb.at[(qb + 2) % 4], esems[b])
                    else:
                        @pl.when(c + 2 < ch)
                        def _():
                            pltpu.async_copy(ed_h.at[cid, sid, c + 2],
                                             eb.at[(qb + 2) % 4], esems[b])
                    if qb < 3:
                        pltpu.make_async_copy(ed_h.at[cid, sid, c + 1],
                                              eb.at[(qb + 1) % 4],
                                              esems[(qb + 1) % 2]).wait()
                        pltpu.async_copy(table_h.at[eb.at[(qb + 1) % 4, 0]],
                                         rows_v.at[(qb + 1) % 4], gsem0)
                    else:
                        @pl.when(c + 1 < ch)
                        def _():
                            pltpu.make_async_copy(ed_h.at[cid, sid, c + 1],
                                                  eb.at[(qb + 1) % 4],
                                                  esems[(qb + 1) % 2]).wait()
                            pltpu.async_copy(
                                table_h.at[eb.at[(qb + 1) % 4, 0]],
                                rows_v.at[(qb + 1) % 4], gsem0)
                return carry

            lax.fori_loop(0, ch // 4, quad, 0)
            for _ in range(2):
                pltpu.make_async_copy(table_h.at[eb.at[0, 0]],
                                      rows_v.at[0], gsem0).wait()
            plsc.subcore_barrier()
            for i in range(n_chunks):
                pltpu.sync_copy(acc.at[pl.ds(base + i * K, K)],
                                o_h.at[cid, pl.ds(base + i * K, K)])
            plsc.subcore_barrier()

        spmm_pass(False, o1_h)
        spmm_pass(True, o2_h)

    return sc_kernel


def _tc_tail(o1, o2, f, w1t, w2t, b1, b2, n, d, half):
    blk = 400
    grid = (n // blk,)

    def body(o1a, o1b, o2a, o2b, fr, w1, w2, bb1, bb2, out):
        agg1 = jnp.concatenate([o1a[...], o1b[...]], axis=1) + fr[...]
        x1 = jnp.dot(agg1, w1[...], preferred_element_type=jnp.float32) + bb1[...]
        agg2 = jnp.concatenate([o2a[...], o2b[...]], axis=1)
        x2 = jnp.dot(agg2, w2[...], preferred_element_type=jnp.float32) + bb2[...]
        y1 = jnp.where(x1 > 0, x1, 0.01 * x1)
        y2 = jnp.where(x2 > 0, x2, 0.01 * x2)
        out[...] = y1 + y2

    hs = pl.BlockSpec((blk, half), lambda i: (i, 0))
    fs = pl.BlockSpec((blk, d), lambda i: (i, 0))
    ws = pl.BlockSpec((d, d), lambda i: (0, 0))
    bs = pl.BlockSpec((1, d), lambda i: (0, 0))
    return pl.pallas_call(
        body,
        grid=grid,
        in_specs=[hs, hs, hs, hs, fs, ws, ws, bs, bs],
        out_specs=fs,
        out_shape=jax.ShapeDtypeStruct((n, d), jnp.float32),
    )(o1[0], o1[1], o2[0], o2[1], f, w1t, w2t, b1, b2)


def kernel(features, edge_row, edge_col, edge_val, W1, b1, W2, b2):
    n, d = features.shape
    e = edge_row.shape[0]
    half = d // 2

    gran = NS * K * 4
    e_pad = -(-e // gran) * gran
    pad = e_pad - e
    ch = e_pad // (NS * K)
    col_p = jnp.pad(edge_col, (0, pad))
    row_p = jnp.pad(edge_row, (0, pad))
    val_p = jnp.pad(edge_val, (0, pad))
    val_bits = jax.lax.bitcast_convert_type(val_p, jnp.int32)
    col2 = jnp.stack([col_p, col_p])
    row2 = jnp.broadcast_to(row_p, (NC, e_pad))
    vb2 = jnp.broadcast_to(val_bits, (NC, e_pad))
    edata = jnp.stack([col2, row2, vb2], axis=1)
    edata = edata.reshape(NC, 3, NS, ch, K).transpose(0, 2, 3, 1, 4)
    table = features

    n_acc = -(-n // (NS * K)) * NS * K

    o1, o2 = _make_sc_spmm(n_acc, half, ch)(table, edata)

    return _tc_tail(o1, o2, features, W1.T, W2.T,
                    b1.reshape(1, d), b2.reshape(1, d), n, d, half)


# speedup vs baseline: 2.1055x; 2.1055x over previous
import functools

import jax
import jax.numpy as jnp
from jax import lax
from jax.experimental import pallas as pl
from jax.experimental.pallas import tpu as pltpu
from jax.experimental.pallas import tpu_sc as plsc

NC = 2
NS = 16
L = 16
K = 128


def _make_sc_spmm(n, half, ch):
    npt = n // NS
    n_chunks = npt // K
    mesh = plsc.VectorSubcoreMesh(
        core_axis_name="c", subcore_axis_name="s", num_cores=NC,
        num_subcores=NS)

    @functools.partial(
        pl.kernel,
        out_type=[
            jax.ShapeDtypeStruct((NC, n, half), jnp.float32),
            jax.ShapeDtypeStruct((NC, n, half), jnp.float32),
        ],
        mesh=mesh,
        scratch_types=[
            pltpu.VMEM((8, 3, K), jnp.int32),
            pltpu.VMEM((4, K, half), jnp.bfloat16),
            pltpu.VMEM((2, K, half), jnp.float32),
            pltpu.VMEM_SHARED((n, half), jnp.float32),
            pltpu.SemaphoreType.DMA,
            pltpu.SemaphoreType.DMA,
            pltpu.SemaphoreType.DMA,
            pltpu.SemaphoreType.DMA,
            pltpu.SemaphoreType.DMA,
            pltpu.SemaphoreType.DMA,
            pltpu.SemaphoreType.DMA,
            pltpu.SemaphoreType.DMA,
        ],
        compiler_params=pltpu.CompilerParams(use_tc_tiling_on_sc=False,
                                             needs_layout_passes=False),
    )
    def sc_kernel(table_h, ed_h, o1_h, o2_h,
                  eb, rows_v, m_v, acc,
                  esem0, esem1, gsem0, gsem1, gsem2, gsem3, ssem0, ssem1):
        cid = lax.axis_index("c")
        sid = lax.axis_index("s")
        esems = (esem0, esem1)
        gsems = (gsem0, gsem1, gsem2, gsem3)
        ssems = (ssem0, ssem1)

        base = sid * npt
        zero = jnp.zeros((L,), jnp.float32)

        def zero_acc():
            def zb(k, carry):
                for j in range(half // L):
                    m_v[0, k, pl.ds(j * L, L)] = zero
                return carry

            lax.fori_loop(0, K, zb, 0)
            for i in range(n_chunks):
                pltpu.sync_copy(m_v.at[0], acc.at[pl.ds(base + i * K, K)])

        def spmm_pass(square, o_h):
            zero_acc()
            plsc.subcore_barrier()

            def compute(q, bq, mq):
                def group(g, carry2):
                    vv = plsc.bitcast(eb[q, 2, pl.ds(g * L, L)], jnp.float32)
                    for k in range(L):
                        v = vv[k]
                        kk = g * L + k
                        for j in range(half // (2 * L)):
                            ab = rows_v[bq, kk, pl.ds(j * 2 * L, 2 * L)]
                            ra, rb = plsc.unpack(
                                ab, format=plsc.PackFormat.INTERLEAVED)
                            ma = ra * v
                            mb = rb * v
                            if square:
                                ma = ma * ra
                                mb = mb * rb
                            m_v[mq, kk, pl.ds(j * 2 * L, L)] = ma
                            m_v[mq, kk, pl.ds(j * 2 * L + L, L)] = mb
                    return carry2

                lax.fori_loop(0, K // L, group, 0)

            pltpu.async_copy(ed_h.at[cid, sid, 0], eb.at[0], esem0)
            pltpu.async_copy(ed_h.at[cid, sid, 1], eb.at[1], esem1)
            pltpu.make_async_copy(ed_h.at[cid, sid, 0], eb.at[0],
                                  esem0).wait()
            pltpu.async_copy(ed_h.at[cid, sid, 2], eb.at[2], esem0)
            pltpu.async_copy(table_h.at[eb.at[0, 0]], rows_v.at[0], gsem0)
            pltpu.make_async_copy(ed_h.at[cid, sid, 1], eb.at[1],
                                  esem1).wait()
            pltpu.async_copy(table_h.at[eb.at[1, 0]], rows_v.at[1], gsem1)

            def octo(p, carry):
                for q in range(8):
                    c = 8 * p + q
                    bq = q % 4
                    mq = q % 2
                    pltpu.make_async_copy(table_h.at[eb.at[q, 0]],
                                          rows_v.at[bq], gsems[bq]).wait()
                    if q < 2:
                        @pl.when(p > 0)
                        def _():
                            pltpu.make_async_copy(
                                m_v.at[mq], acc.at[eb.at[q, 1]],
                                ssems[mq]).wait()
                    else:
                        pltpu.make_async_copy(m_v.at[mq],
                                              acc.at[eb.at[q, 1]],
                                              ssems[mq]).wait()
                    if q < 5:
                        pltpu.async_copy(ed_h.at[cid, sid, c + 3],
                                         eb.at[(q + 3) % 8],
                                         esems[(q + 1) % 2])
                    else:
                        @pl.when(c + 3 < ch)
                        def _():
                            pltpu.async_copy(ed_h.at[cid, sid, c + 3],
                                             eb.at[(q + 3) % 8],
                                             esems[(q + 1) % 2])
                    if q < 6:
                        pltpu.make_async_copy(ed_h.at[cid, sid, c + 2],
                                              eb.at[(q + 2) % 8],
                                              esems[q % 2]).wait()
                        pltpu.async_copy(table_h.at[eb.at[(q + 2) % 8, 0]],
                                         rows_v.at[(q + 2) % 4],
                                         gsems[(q + 2) % 4])
                    else:
                        @pl.when(c + 2 < ch)
                        def _():
                            pltpu.make_async_copy(ed_h.at[cid, sid, c + 2],
                                                  eb.at[(q + 2) % 8],
                                                  esems[q % 2]).wait()
                            pltpu.async_copy(
                                table_h.at[eb.at[(q + 2) % 8, 0]],
                                rows_v.at[(q + 2) % 4], gsems[(q + 2) % 4])
                    compute(q, bq, mq)
                    pltpu.async_copy(m_v.at[mq], acc.at[eb.at[q, 1]],
                                     ssems[mq], add=True)
                return carry

            lax.fori_loop(0, ch // 8, octo, 0)
            pltpu.make_async_copy(m_v.at[0], acc.at[eb.at[6, 1]],
                                  ssems[0]).wait()
            pltpu.make_async_copy(m_v.at[1], acc.at[eb.at[7, 1]],
                                  ssems[1]).wait()
            plsc.subcore_barrier()
            for i in range(n_chunks):
                pltpu.sync_copy(acc.at[pl.ds(base + i * K, K)],
                                o_h.at[cid, pl.ds(base + i * K, K)])
            plsc.subcore_barrier()

        spmm_pass(False, o1_h)
        spmm_pass(True, o2_h)

    return sc_kernel


def _tc_tail(o1, o2, f, w1t, w2t, b1, b2, n, d, half):
    blk = 400
    grid = (n // blk,)

    def body(o1a, o1b, o2a, o2b, fr, w1, w2, bb1, bb2, out):
        agg1 = jnp.concatenate([o1a[...], o1b[...]], axis=1) + fr[...]
        x1 = jnp.dot(agg1, w1[...], preferred_element_type=jnp.float32) + bb1[...]
        agg2 = jnp.concatenate([o2a[...], o2b[...]], axis=1)
        x2 = jnp.dot(agg2, w2[...], preferred_element_type=jnp.float32) + bb2[...]
        y1 = jnp.where(x1 > 0, x1, 0.01 * x1)
        y2 = jnp.where(x2 > 0, x2, 0.01 * x2)
        out[...] = y1 + y2

    hs = pl.BlockSpec((blk, half), lambda i: (i, 0))
    fs = pl.BlockSpec((blk, d), lambda i: (i, 0))
    ws = pl.BlockSpec((d, d), lambda i: (0, 0))
    bs = pl.BlockSpec((1, d), lambda i: (0, 0))
    return pl.pallas_call(
        body,
        grid=grid,
        in_specs=[hs, hs, hs, hs, fs, ws, ws, bs, bs],
        out_specs=fs,
        out_shape=jax.ShapeDtypeStruct((n, d), jnp.float32),
    )(o1[0], o1[1], o2[0], o2[1], f, w1t, w2t, b1, b2)


def kernel(features, edge_row, edge_col, edge_val, W1, b1, W2, b2):
    n, d = features.shape
    e = edge_row.shape[0]
    half = d // 2

    gran = NS * K * 8
    e_pad = -(-e // gran) * gran
    pad = e_pad - e
    ch = e_pad // (NS * K)
    col_p = jnp.pad(edge_col, (0, pad))
    row_p = jnp.pad(edge_row, (0, pad))
    val_p = jnp.pad(edge_val, (0, pad))
    val_bits = jax.lax.bitcast_convert_type(val_p, jnp.int32)
    col2 = jnp.stack([col_p, col_p + n])
    row2 = jnp.broadcast_to(row_p, (NC, e_pad))
    vb2 = jnp.broadcast_to(val_bits, (NC, e_pad))
    edata = jnp.stack([col2, row2, vb2], axis=1)
    edata = edata.reshape(NC, 3, NS, ch, K).transpose(0, 2, 3, 1, 4)
    table = features.reshape(n, NC, half).transpose(1, 0, 2)
    table = table.reshape(NC * n, half).astype(jnp.bfloat16)
    table = table.reshape(NC * n, half // (2 * L), 2, L)
    table = table.transpose(0, 1, 3, 2).reshape(NC * n, half)

    n_acc = -(-n // (NS * K)) * NS * K

    o1, o2 = _make_sc_spmm(n_acc, half, ch)(table, edata)

    return _tc_tail(o1, o2, features, W1.T, W2.T,
                    b1.reshape(1, d), b2.reshape(1, d), n, d, half)


# speedup vs baseline: 2.8603x; 1.3585x over previous
import functools

import jax
import jax.numpy as jnp
import numpy as np
from jax import lax
from jax.experimental import pallas as pl
from jax.experimental.pallas import tpu as pltpu
from jax.experimental.pallas import tpu_sc as plsc

NC = 2
NS = 16
L = 16
K = 128


def _make_sc_spmm(n, half, ch):
    npt = n // NS
    n_chunks = npt // K
    mesh = plsc.VectorSubcoreMesh(
        core_axis_name="c", subcore_axis_name="s", num_cores=NC,
        num_subcores=NS)

    @functools.partial(
        pl.kernel,
        out_type=[
            jax.ShapeDtypeStruct((NC, n, half), jnp.float32),
            jax.ShapeDtypeStruct((NC, n, half), jnp.bfloat16),
        ],
        mesh=mesh,
        scratch_types=[
            pltpu.VMEM((8, 3, K), jnp.int32),
            pltpu.VMEM((4, K, half), jnp.float32),
            pltpu.VMEM((2, K, half), jnp.float32),
            pltpu.VMEM((2, K, half), jnp.bfloat16),
            pltpu.VMEM_SHARED((n, half), jnp.float32),
            pltpu.VMEM_SHARED((n, half), jnp.bfloat16),
            pltpu.SemaphoreType.DMA,
            pltpu.SemaphoreType.DMA,
            pltpu.SemaphoreType.DMA,
            pltpu.SemaphoreType.DMA,
            pltpu.SemaphoreType.DMA,
            pltpu.SemaphoreType.DMA,
            pltpu.SemaphoreType.DMA,
            pltpu.SemaphoreType.DMA,
            pltpu.SemaphoreType.DMA,
            pltpu.SemaphoreType.DMA,
        ],
        compiler_params=pltpu.CompilerParams(use_tc_tiling_on_sc=False,
                                             needs_layout_passes=False),
    )
    def sc_kernel(table_h, ed_h, o1_h, o2_h,
                  eb, rows_v, m1_v, m2_v, acc1, acc2,
                  esem0, esem1, gsem0, gsem1, gsem2, gsem3,
                  s1sem0, s1sem1, s2sem0, s2sem1):
        cid = lax.axis_index("c")
        sid = lax.axis_index("s")
        esems = (esem0, esem1)
        gsems = (gsem0, gsem1, gsem2, gsem3)
        s1sems = (s1sem0, s1sem1)
        s2sems = (s2sem0, s2sem1)

        base = sid * npt
        zero = jnp.zeros((L,), jnp.float32)
        zero2 = jnp.zeros((2 * L,), jnp.bfloat16)

        def zb(k, carry):
            for j in range(half // L):
                m1_v[0, k, pl.ds(j * L, L)] = zero
            for j in range(half // (2 * L)):
                m2_v[0, k, pl.ds(j * 2 * L, 2 * L)] = zero2
            return carry

        lax.fori_loop(0, K, zb, 0)
        for i in range(n_chunks):
            pltpu.sync_copy(m1_v.at[0], acc1.at[pl.ds(base + i * K, K)])
            pltpu.sync_copy(m2_v.at[0], acc2.at[pl.ds(base + i * K, K)])
        plsc.subcore_barrier()

        def compute(q, bq, mq):
            def group(g, carry2):
                vv = plsc.bitcast(eb[q, 2, pl.ds(g * L, L)], jnp.float32)
                for k in range(L):
                    v = vv[k]
                    kk = g * L + k
                    for j in range(half // (2 * L)):
                        ra = rows_v[bq, kk, pl.ds(j * 2 * L, L)]
                        rb = rows_v[bq, kk, pl.ds(j * 2 * L + L, L)]
                        m1a = ra * v
                        m1b = rb * v
                        m1_v[mq, kk, pl.ds(j * 2 * L, L)] = m1a
                        m1_v[mq, kk, pl.ds(j * 2 * L + L, L)] = m1b
                        m2_v[mq, kk, pl.ds(j * 2 * L, 2 * L)] = plsc.pack(
                            m1a * ra, m1b * rb,
                            format=plsc.PackFormat.INTERLEAVED)
                return carry2

            lax.fori_loop(0, K // L, group, 0)

        pltpu.async_copy(ed_h.at[cid, sid, 0], eb.at[0], esem0)
        pltpu.async_copy(ed_h.at[cid, sid, 1], eb.at[1], esem1)
        pltpu.make_async_copy(ed_h.at[cid, sid, 0], eb.at[0], esem0).wait()
        pltpu.async_copy(ed_h.at[cid, sid, 2], eb.at[2], esem0)
        pltpu.async_copy(table_h.at[eb.at[0, 0]], rows_v.at[0], gsem0)
        pltpu.make_async_copy(ed_h.at[cid, sid, 1], eb.at[1], esem1).wait()
        pltpu.async_copy(table_h.at[eb.at[1, 0]], rows_v.at[1], gsem1)

        def octo(p, carry):
            for q in range(8):
                c = 8 * p + q
                bq = q % 4
                mq = q % 2
                pltpu.make_async_copy(table_h.at[eb.at[q, 0]],
                                      rows_v.at[bq], gsems[bq]).wait()
                if q < 2:
                    @pl.when(p > 0)
                    def _():
                        pltpu.make_async_copy(m1_v.at[mq],
                                              acc1.at[eb.at[q, 1]],
                                              s1sems[mq]).wait()
                        pltpu.make_async_copy(m2_v.at[mq],
                                              acc2.at[eb.at[q, 1]],
                                              s2sems[mq]).wait()
                else:
                    pltpu.make_async_copy(m1_v.at[mq],
                                          acc1.at[eb.at[q, 1]],
                                          s1sems[mq]).wait()
                    pltpu.make_async_copy(m2_v.at[mq],
                                          acc2.at[eb.at[q, 1]],
                                          s2sems[mq]).wait()
                if q < 5:
                    pltpu.async_copy(ed_h.at[cid, sid, c + 3],
                                     eb.at[(q + 3) % 8], esems[(q + 1) % 2])
                else:
                    @pl.when(c + 3 < ch)
                    def _():
                        pltpu.async_copy(ed_h.at[cid, sid, c + 3],
                                         eb.at[(q + 3) % 8],
                                         esems[(q + 1) % 2])
                if q < 6:
                    pltpu.make_async_copy(ed_h.at[cid, sid, c + 2],
                                          eb.at[(q + 2) % 8],
                                          esems[q % 2]).wait()
                    pltpu.async_copy(table_h.at[eb.at[(q + 2) % 8, 0]],
                                     rows_v.at[(q + 2) % 4],
                                     gsems[(q + 2) % 4])
                else:
                    @pl.when(c + 2 < ch)
                    def _():
                        pltpu.make_async_copy(ed_h.at[cid, sid, c + 2],
                                              eb.at[(q + 2) % 8],
                                              esems[q % 2]).wait()
                        pltpu.async_copy(table_h.at[eb.at[(q + 2) % 8, 0]],
                                         rows_v.at[(q + 2) % 4],
                                         gsems[(q + 2) % 4])
                compute(q, bq, mq)
                pltpu.async_copy(m1_v.at[mq], acc1.at[eb.at[q, 1]],
                                 s1sems[mq], add=True)
                pltpu.async_copy(m2_v.at[mq], acc2.at[eb.at[q, 1]],
                                 s2sems[mq], add=True)
            return carry

        lax.fori_loop(0, ch // 8, octo, 0)
        pltpu.make_async_copy(m1_v.at[0], acc1.at[eb.at[6, 1]],
                              s1sems[0]).wait()
        pltpu.make_async_copy(m1_v.at[1], acc1.at[eb.at[7, 1]],
                              s1sems[1]).wait()
        pltpu.make_async_copy(m2_v.at[0], acc2.at[eb.at[6, 1]],
                              s2sems[0]).wait()
        pltpu.make_async_copy(m2_v.at[1], acc2.at[eb.at[7, 1]],
                              s2sems[1]).wait()
        plsc.subcore_barrier()
        for i in range(n_chunks):
            pltpu.sync_copy(acc1.at[pl.ds(base + i * K, K)],
                            o1_h.at[cid, pl.ds(base + i * K, K)])
            pltpu.sync_copy(acc2.at[pl.ds(base + i * K, K)],
                            o2_h.at[cid, pl.ds(base + i * K, K)])
        plsc.subcore_barrier()

    return sc_kernel


def _tc_tail(o1, o2, f, w1t, w2p, b1, b2, n, d, half):
    blk = 400
    grid = (n // blk,)

    def body(o1a, o1b, o2a, o2b, fr, w1, w2, bb1, bb2, out):
        agg1 = jnp.concatenate([o1a[...], o1b[...]], axis=1) + fr[...]
        x1 = jnp.dot(agg1, w1[...], preferred_element_type=jnp.float32) + bb1[...]
        agg2 = jnp.concatenate([o2a[...], o2b[...]],
                               axis=1).astype(jnp.float32)
        x2 = jnp.dot(agg2, w2[...], preferred_element_type=jnp.float32) + bb2[...]
        y1 = jnp.where(x1 > 0, x1, 0.01 * x1)
        y2 = jnp.where(x2 > 0, x2, 0.01 * x2)
        out[...] = y1 + y2

    hs = pl.BlockSpec((blk, half), lambda i: (i, 0))
    fs = pl.BlockSpec((blk, d), lambda i: (i, 0))
    ws = pl.BlockSpec((d, d), lambda i: (0, 0))
    bs = pl.BlockSpec((1, d), lambda i: (0, 0))
    return pl.pallas_call(
        body,
        grid=grid,
        in_specs=[hs, hs, hs, hs, fs, ws, ws, bs, bs],
        out_specs=fs,
        out_shape=jax.ShapeDtypeStruct((n, d), jnp.float32),
    )(o1[0], o1[1], o2[0], o2[1], f, w1t, w2p, b1, b2)


def kernel(features, edge_row, edge_col, edge_val, W1, b1, W2, b2):
    n, d = features.shape
    e = edge_row.shape[0]
    half = d // 2

    gran = NS * K * 8
    e_pad = -(-e // gran) * gran
    pad = e_pad - e
    ch = e_pad // (NS * K)
    col_p = jnp.pad(edge_col, (0, pad))
    row_p = jnp.pad(edge_row, (0, pad))
    val_p = jnp.pad(edge_val, (0, pad))
    val_bits = jax.lax.bitcast_convert_type(val_p, jnp.int32)
    col2 = jnp.stack([col_p, col_p + n])
    row2 = jnp.broadcast_to(row_p, (NC, e_pad))
    vb2 = jnp.broadcast_to(val_bits, (NC, e_pad))
    edata = jnp.stack([col2, row2, vb2], axis=1)
    edata = edata.reshape(NC, 3, NS, ch, K).transpose(0, 2, 3, 1, 4)
    table = features.reshape(n, NC, half).transpose(1, 0, 2)
    table = table.reshape(NC * n, half)

    n_acc = -(-n // (NS * K)) * NS * K

    o1, o2 = _make_sc_spmm(n_acc, half, ch)(table, edata)

    perm = np.arange(d).reshape(d // (2 * L), 2, L).transpose(0, 2, 1)
    perm = perm.reshape(d)
    w2p = W2.T[perm]

    return _tc_tail(o1, o2, features, W1.T, w2p,
                    b1.reshape(1, d), b2.reshape(1, d), n, d, half)


# speedup vs baseline: 2.8761x; 1.0055x over previous
import functools

import jax
import jax.numpy as jnp
import numpy as np
from jax import lax
from jax.experimental import pallas as pl
from jax.experimental.pallas import tpu as pltpu
from jax.experimental.pallas import tpu_sc as plsc

NC = 2
NS = 16
L = 16
K = 128


def _make_sc_spmm(n, half, ch):
    npt = n // NS
    n_chunks = npt // K
    mesh = plsc.VectorSubcoreMesh(
        core_axis_name="c", subcore_axis_name="s", num_cores=NC,
        num_subcores=NS)

    @functools.partial(
        pl.kernel,
        out_type=[
            jax.ShapeDtypeStruct((NC, n, half), jnp.float32),
            jax.ShapeDtypeStruct((NC, n, half), jnp.bfloat16),
        ],
        mesh=mesh,
        scratch_types=[
            pltpu.VMEM((8, 3, K), jnp.int32),
            pltpu.VMEM((4, K, half), jnp.float32),
            pltpu.VMEM((2, K, half), jnp.float32),
            pltpu.VMEM((2, K, half), jnp.bfloat16),
            pltpu.VMEM_SHARED((n, half), jnp.float32),
            pltpu.VMEM_SHARED((n, half), jnp.bfloat16),
            pltpu.SemaphoreType.DMA,
            pltpu.SemaphoreType.DMA,
            pltpu.SemaphoreType.DMA,
            pltpu.SemaphoreType.DMA,
            pltpu.SemaphoreType.DMA,
            pltpu.SemaphoreType.DMA,
            pltpu.SemaphoreType.DMA,
            pltpu.SemaphoreType.DMA,
            pltpu.SemaphoreType.DMA,
            pltpu.SemaphoreType.DMA,
        ],
        compiler_params=pltpu.CompilerParams(use_tc_tiling_on_sc=False,
                                             needs_layout_passes=False),
    )
    def sc_kernel(table_h, ed_h, o1_h, o2_h,
                  eb, rows_v, m1_v, m2_v, acc1, acc2,
                  esem0, esem1, gsem0, gsem1, gsem2, gsem3,
                  s1sem0, s1sem1, s2sem0, s2sem1):
        cid = lax.axis_index("c")
        sid = lax.axis_index("s")
        esems = (esem0, esem1)
        gsems = (gsem0, gsem1, gsem2, gsem3)
        s1sems = (s1sem0, s1sem1)
        s2sems = (s2sem0, s2sem1)

        base = sid * npt
        zero = jnp.zeros((L,), jnp.float32)
        zero2 = jnp.zeros((2 * L,), jnp.bfloat16)

        def zb(k, carry):
            for j in range(half // L):
                m1_v[0, k, pl.ds(j * L, L)] = zero
            for j in range(half // (2 * L)):
                m2_v[0, k, pl.ds(j * 2 * L, 2 * L)] = zero2
            return carry

        lax.fori_loop(0, K, zb, 0)
        for i in range(n_chunks):
            pltpu.sync_copy(m1_v.at[0], acc1.at[pl.ds(base + i * K, K)])
            pltpu.sync_copy(m2_v.at[0], acc2.at[pl.ds(base + i * K, K)])
        plsc.subcore_barrier()

        def compute(q, bq, mq):
            def group(g, carry2):
                vv = plsc.bitcast(eb[q, 2, pl.ds(g * L, L)], jnp.float32)
                for k in range(L):
                    v = vv[k]
                    kk = g * L + k
                    for j in range(half // (2 * L)):
                        ra = rows_v[bq, kk, pl.ds(j * 2 * L, L)]
                        rb = rows_v[bq, kk, pl.ds(j * 2 * L + L, L)]
                        m1a = ra * v
                        m1b = rb * v
                        m1_v[mq, kk, pl.ds(j * 2 * L, L)] = m1a
                        m1_v[mq, kk, pl.ds(j * 2 * L + L, L)] = m1b
                        m2_v[mq, kk, pl.ds(j * 2 * L, 2 * L)] = plsc.pack(
                            m1a * ra, m1b * rb,
                            format=plsc.PackFormat.INTERLEAVED)
                return carry2

            lax.fori_loop(0, K // L, group, 0)

        pltpu.async_copy(ed_h.at[cid, sid, 0], eb.at[0], esem0)
        pltpu.async_copy(ed_h.at[cid, sid, 1], eb.at[1], esem1)
        pltpu.make_async_copy(ed_h.at[cid, sid, 0], eb.at[0], esem0).wait()
        pltpu.async_copy(ed_h.at[cid, sid, 2], eb.at[2], esem0)
        pltpu.async_copy(table_h.at[eb.at[0, 0]], rows_v.at[0], gsem0)
        pltpu.make_async_copy(ed_h.at[cid, sid, 1], eb.at[1], esem1).wait()
        pltpu.async_copy(ed_h.at[cid, sid, 3], eb.at[3], esem1)
        pltpu.async_copy(table_h.at[eb.at[1, 0]], rows_v.at[1], gsem1)
        pltpu.make_async_copy(ed_h.at[cid, sid, 2], eb.at[2], esem0).wait()
        pltpu.async_copy(ed_h.at[cid, sid, 4], eb.at[4], esem0)
        pltpu.async_copy(table_h.at[eb.at[2, 0]], rows_v.at[2], gsem2)

        def octo(p, carry):
            for q in range(8):
                c = 8 * p + q
                bq = q % 4
                mq = q % 2
                pltpu.make_async_copy(table_h.at[eb.at[q, 0]],
                                      rows_v.at[bq], gsems[bq]).wait()
                if q < 2:
                    @pl.when(p > 0)
                    def _():
                        pltpu.make_async_copy(m1_v.at[mq],
                                              acc1.at[eb.at[q, 1]],
                                              s1sems[mq]).wait()
                        pltpu.make_async_copy(m2_v.at[mq],
                                              acc2.at[eb.at[q, 1]],
                                              s2sems[mq]).wait()
                else:
                    pltpu.make_async_copy(m1_v.at[mq],
                                          acc1.at[eb.at[q, 1]],
                                          s1sems[mq]).wait()
                    pltpu.make_async_copy(m2_v.at[mq],
                                          acc2.at[eb.at[q, 1]],
                                          s2sems[mq]).wait()
                if q < 5:
                    pltpu.make_async_copy(ed_h.at[cid, sid, c + 3],
                                          eb.at[(q + 3) % 8],
                                          esems[(q + 1) % 2]).wait()
                    pltpu.async_copy(table_h.at[eb.at[(q + 3) % 8, 0]],
                                     rows_v.at[(q + 3) % 4],
                                     gsems[(q + 3) % 4])
                else:
                    @pl.when(c + 3 < ch)
                    def _():
                        pltpu.make_async_copy(ed_h.at[cid, sid, c + 3],
                                              eb.at[(q + 3) % 8],
                                              esems[(q + 1) % 2]).wait()
                        pltpu.async_copy(table_h.at[eb.at[(q + 3) % 8, 0]],
                                         rows_v.at[(q + 3) % 4],
                                         gsems[(q + 3) % 4])
                if q < 3:
                    pltpu.async_copy(ed_h.at[cid, sid, c + 5],
                                     eb.at[(q + 5) % 8], esems[(q + 1) % 2])
                else:
                    @pl.when(c + 5 < ch)
                    def _():
                        pltpu.async_copy(ed_h.at[cid, sid, c + 5],
                                         eb.at[(q + 5) % 8],
                                         esems[(q + 1) % 2])
                compute(q, bq, mq)
                pltpu.async_copy(m1_v.at[mq], acc1.at[eb.at[q, 1]],
                                 s1sems[mq], add=True)
                pltpu.async_copy(m2_v.at[mq], acc2.at[eb.at[q, 1]],
                                 s2sems[mq], add=True)
            return carry

        lax.fori_loop(0, ch // 8, octo, 0)
        pltpu.make_async_copy(m1_v.at[0], acc1.at[eb.at[6, 1]],
                              s1sems[0]).wait()
        pltpu.make_async_copy(m1_v.at[1], acc1.at[eb.at[7, 1]],
                              s1sems[1]).wait()
        pltpu.make_async_copy(m2_v.at[0], acc2.at[eb.at[6, 1]],
                              s2sems[0]).wait()
        pltpu.make_async_copy(m2_v.at[1], acc2.at[eb.at[7, 1]],
                              s2sems[1]).wait()
        plsc.subcore_barrier()
        for i in range(n_chunks):
            pltpu.sync_copy(acc1.at[pl.ds(base + i * K, K)],
                            o1_h.at[cid, pl.ds(base + i * K, K)])
            pltpu.sync_copy(acc2.at[pl.ds(base + i * K, K)],
                            o2_h.at[cid, pl.ds(base + i * K, K)])
        plsc.subcore_barrier()

    return sc_kernel


def _tc_tail(o1, o2, f, w1t, w2p, b1, b2, n, d, half):
    blk = 400
    grid = (n // blk,)

    def body(o1a, o1b, o2a, o2b, fr, w1, w2, bb1, bb2, out):
        agg1 = jnp.concatenate([o1a[...], o1b[...]], axis=1) + fr[...]
        x1 = jnp.dot(agg1, w1[...], preferred_element_type=jnp.float32) + bb1[...]
        agg2 = jnp.concatenate([o2a[...], o2b[...]],
                               axis=1).astype(jnp.float32)
        x2 = jnp.dot(agg2, w2[...], preferred_element_type=jnp.float32) + bb2[...]
        y1 = jnp.where(x1 > 0, x1, 0.01 * x1)
        y2 = jnp.where(x2 > 0, x2, 0.01 * x2)
        out[...] = y1 + y2

    hs = pl.BlockSpec((blk, half), lambda i: (i, 0))
    fs = pl.BlockSpec((blk, d), lambda i: (i, 0))
    ws = pl.BlockSpec((d, d), lambda i: (0, 0))
    bs = pl.BlockSpec((1, d), lambda i: (0, 0))
    return pl.pallas_call(
        body,
        grid=grid,
        in_specs=[hs, hs, hs, hs, fs, ws, ws, bs, bs],
        out_specs=fs,
        out_shape=jax.ShapeDtypeStruct((n, d), jnp.float32),
    )(o1[0], o1[1], o2[0], o2[1], f, w1t, w2p, b1, b2)


def kernel(features, edge_row, edge_col, edge_val, W1, b1, W2, b2):
    n, d = features.shape
    e = edge_row.shape[0]
    half = d // 2

    gran = NS * K * 8
    e_pad = -(-e // gran) * gran
    pad = e_pad - e
    ch = e_pad // (NS * K)
    col_p = jnp.pad(edge_col, (0, pad))
    row_p = jnp.pad(edge_row, (0, pad))
    val_p = jnp.pad(edge_val, (0, pad))
    val_bits = jax.lax.bitcast_convert_type(val_p, jnp.int32)
    col2 = jnp.stack([col_p, col_p + n])
    row2 = jnp.broadcast_to(row_p, (NC, e_pad))
    vb2 = jnp.broadcast_to(val_bits, (NC, e_pad))
    edata = jnp.stack([col2, row2, vb2], axis=1)
    edata = edata.reshape(NC, 3, NS, ch, K).transpose(0, 2, 3, 1, 4)
    table = features.reshape(n, NC, half).transpose(1, 0, 2)
    table = table.reshape(NC * n, half)

    n_acc = -(-n // (NS * K)) * NS * K

    o1, o2 = _make_sc_spmm(n_acc, half, ch)(table, edata)

    perm = np.arange(d).reshape(d // (2 * L), 2, L).transpose(0, 2, 1)
    perm = perm.reshape(d)
    w2p = W2.T[perm]

    return _tc_tail(o1, o2, features, W1.T, w2p,
                    b1.reshape(1, d), b2.reshape(1, d), n, d, half)


# speedup vs baseline: 3.6790x; 1.2792x over previous
import functools

import jax
import jax.numpy as jnp
import numpy as np
from jax import lax
from jax.experimental import pallas as pl
from jax.experimental.pallas import tpu as pltpu
from jax.experimental.pallas import tpu_sc as plsc

NC = 2
NS = 16
L = 16
K = 128


def _make_sc_spmm(n, half, ch):
    npt = n // NS
    n_chunks = npt // K
    mesh = plsc.VectorSubcoreMesh(
        core_axis_name="c", subcore_axis_name="s", num_cores=NC,
        num_subcores=NS)

    @functools.partial(
        pl.kernel,
        out_type=[
            jax.ShapeDtypeStruct((NC, n, half), jnp.float32),
            jax.ShapeDtypeStruct((NC, n, half), jnp.bfloat16),
        ],
        mesh=mesh,
        scratch_types=[
            pltpu.VMEM((8, 3, K), jnp.int32),
            pltpu.VMEM((4, K, half), jnp.bfloat16),
            pltpu.VMEM((2, K, half), jnp.float32),
            pltpu.VMEM((2, K, half), jnp.bfloat16),
            pltpu.VMEM_SHARED((n, half), jnp.float32),
            pltpu.VMEM_SHARED((n, half), jnp.bfloat16),
            pltpu.SemaphoreType.DMA,
            pltpu.SemaphoreType.DMA,
            pltpu.SemaphoreType.DMA,
            pltpu.SemaphoreType.DMA,
            pltpu.SemaphoreType.DMA,
            pltpu.SemaphoreType.DMA,
            pltpu.SemaphoreType.DMA,
            pltpu.SemaphoreType.DMA,
            pltpu.SemaphoreType.DMA,
            pltpu.SemaphoreType.DMA,
        ],
        compiler_params=pltpu.CompilerParams(use_tc_tiling_on_sc=False,
                                             needs_layout_passes=False),
    )
    def sc_kernel(table_h, ed_h, o1_h, o2_h,
                  eb, rows_v, m1_v, m2_v, acc1, acc2,
                  esem0, esem1, gsem0, gsem1, gsem2, gsem3,
                  s1sem0, s1sem1, s2sem0, s2sem1):
        cid = lax.axis_index("c")
        sid = lax.axis_index("s")
        esems = (esem0, esem1)
        gsems = (gsem0, gsem1, gsem2, gsem3)
        s1sems = (s1sem0, s1sem1)
        s2sems = (s2sem0, s2sem1)

        base = sid * npt
        zero = jnp.zeros((L,), jnp.float32)
        zero2 = jnp.zeros((2 * L,), jnp.bfloat16)

        def zb(k, carry):
            for j in range(half // L):
                m1_v[0, k, pl.ds(j * L, L)] = zero
            for j in range(half // (2 * L)):
                m2_v[0, k, pl.ds(j * 2 * L, 2 * L)] = zero2
            return carry

        lax.fori_loop(0, K, zb, 0)
        for i in range(n_chunks):
            pltpu.sync_copy(m1_v.at[0], acc1.at[pl.ds(base + i * K, K)])
            pltpu.sync_copy(m2_v.at[0], acc2.at[pl.ds(base + i * K, K)])
        plsc.subcore_barrier()

        def compute(q, bq, mq):
            def group(g, carry2):
                vv = plsc.bitcast(eb[q, 2, pl.ds(g * L, L)], jnp.float32)
                for k in range(L):
                    v = vv[k]
                    kk = g * L + k
                    for j in range(half // (2 * L)):
                        ab = rows_v[bq, kk, pl.ds(j * 2 * L, 2 * L)]
                        ra, rb = plsc.unpack(
                            ab, format=plsc.PackFormat.INTERLEAVED)
                        m1a = ra * v
                        m1b = rb * v
                        m1_v[mq, kk, pl.ds(j * 2 * L, L)] = m1a
                        m1_v[mq, kk, pl.ds(j * 2 * L + L, L)] = m1b
                        m2_v[mq, kk, pl.ds(j * 2 * L, 2 * L)] = plsc.pack(
                            m1a * ra, m1b * rb,
                            format=plsc.PackFormat.INTERLEAVED)
                return carry2

            lax.fori_loop(0, K // L, group, 0)

        pltpu.async_copy(ed_h.at[cid, sid, 0], eb.at[0], esem0)
        pltpu.async_copy(ed_h.at[cid, sid, 1], eb.at[1], esem1)
        pltpu.make_async_copy(ed_h.at[cid, sid, 0], eb.at[0], esem0).wait()
        pltpu.async_copy(ed_h.at[cid, sid, 2], eb.at[2], esem0)
        pltpu.async_copy(table_h.at[eb.at[0, 0]], rows_v.at[0], gsem0)
        pltpu.make_async_copy(ed_h.at[cid, sid, 1], eb.at[1], esem1).wait()
        pltpu.async_copy(ed_h.at[cid, sid, 3], eb.at[3], esem1)
        pltpu.async_copy(table_h.at[eb.at[1, 0]], rows_v.at[1], gsem1)
        pltpu.make_async_copy(ed_h.at[cid, sid, 2], eb.at[2], esem0).wait()
        pltpu.async_copy(ed_h.at[cid, sid, 4], eb.at[4], esem0)
        pltpu.async_copy(table_h.at[eb.at[2, 0]], rows_v.at[2], gsem2)

        def octo(p, carry):
            for q in range(8):
                c = 8 * p + q
                bq = q % 4
                mq = q % 2
                pltpu.make_async_copy(table_h.at[eb.at[q, 0]],
                                      rows_v.at[bq], gsems[bq]).wait()
                if q < 2:
                    @pl.when(p > 0)
                    def _():
                        pltpu.make_async_copy(m1_v.at[mq],
                                              acc1.at[eb.at[q, 1]],
                                              s1sems[mq]).wait()
                        pltpu.make_async_copy(m2_v.at[mq],
                                              acc2.at[eb.at[q, 1]],
                                              s2sems[mq]).wait()
                else:
                    pltpu.make_async_copy(m1_v.at[mq],
                                          acc1.at[eb.at[q, 1]],
                                          s1sems[mq]).wait()
                    pltpu.make_async_copy(m2_v.at[mq],
                                          acc2.at[eb.at[q, 1]],
                                          s2sems[mq]).wait()
                if q < 5:
                    pltpu.make_async_copy(ed_h.at[cid, sid, c + 3],
                                          eb.at[(q + 3) % 8],
                                          esems[(q + 1) % 2]).wait()
                    pltpu.async_copy(table_h.at[eb.at[(q + 3) % 8, 0]],
                                     rows_v.at[(q + 3) % 4],
                                     gsems[(q + 3) % 4])
                else:
                    @pl.when(c + 3 < ch)
                    def _():
                        pltpu.make_async_copy(ed_h.at[cid, sid, c + 3],
                                              eb.at[(q + 3) % 8],
                                              esems[(q + 1) % 2]).wait()
                        pltpu.async_copy(table_h.at[eb.at[(q + 3) % 8, 0]],
                                         rows_v.at[(q + 3) % 4],
                                         gsems[(q + 3) % 4])
                if q < 3:
                    pltpu.async_copy(ed_h.at[cid, sid, c + 5],
                                     eb.at[(q + 5) % 8], esems[(q + 1) % 2])
                else:
                    @pl.when(c + 5 < ch)
                    def _():
                        pltpu.async_copy(ed_h.at[cid, sid, c + 5],
                                         eb.at[(q + 5) % 8],
                                         esems[(q + 1) % 2])
                compute(q, bq, mq)
                pltpu.async_copy(m1_v.at[mq], acc1.at[eb.at[q, 1]],
                                 s1sems[mq], add=True)
                pltpu.async_copy(m2_v.at[mq], acc2.at[eb.at[q, 1]],
                                 s2sems[mq], add=True)
            return carry

        lax.fori_loop(0, ch // 8, octo, 0)
        pltpu.make_async_copy(m1_v.at[0], acc1.at[eb.at[6, 1]],
                              s1sems[0]).wait()
        pltpu.make_async_copy(m1_v.at[1], acc1.at[eb.at[7, 1]],
                              s1sems[1]).wait()
        pltpu.make_async_copy(m2_v.at[0], acc2.at[eb.at[6, 1]],
                              s2sems[0]).wait()
        pltpu.make_async_copy(m2_v.at[1], acc2.at[eb.at[7, 1]],
                              s2sems[1]).wait()
        plsc.subcore_barrier()
        for i in range(n_chunks):
            pltpu.sync_copy(acc1.at[pl.ds(base + i * K, K)],
                            o1_h.at[cid, pl.ds(base + i * K, K)])
            pltpu.sync_copy(acc2.at[pl.ds(base + i * K, K)],
                            o2_h.at[cid, pl.ds(base + i * K, K)])
        plsc.subcore_barrier()

    return sc_kernel


def _tc_tail(o1, o2, f, w1t, w2p, b1, b2, n, d, half):
    blk = 400
    grid = (n // blk,)

    def body(o1a, o1b, o2a, o2b, fr, w1, w2, bb1, bb2, out):
        agg1 = jnp.concatenate([o1a[...], o1b[...]], axis=1) + fr[...]
        x1 = jnp.dot(agg1, w1[...], preferred_element_type=jnp.float32) + bb1[...]
        agg2 = jnp.concatenate([o2a[...], o2b[...]],
                               axis=1).astype(jnp.float32)
        x2 = jnp.dot(agg2, w2[...], preferred_element_type=jnp.float32) + bb2[...]
        y1 = jnp.where(x1 > 0, x1, 0.01 * x1)
        y2 = jnp.where(x2 > 0, x2, 0.01 * x2)
        out[...] = y1 + y2

    hs = pl.BlockSpec((blk, half), lambda i: (i, 0))
    fs = pl.BlockSpec((blk, d), lambda i: (i, 0))
    ws = pl.BlockSpec((d, d), lambda i: (0, 0))
    bs = pl.BlockSpec((1, d), lambda i: (0, 0))
    return pl.pallas_call(
        body,
        grid=grid,
        in_specs=[hs, hs, hs, hs, fs, ws, ws, bs, bs],
        out_specs=fs,
        out_shape=jax.ShapeDtypeStruct((n, d), jnp.float32),
    )(o1[0], o1[1], o2[0], o2[1], f, w1t, w2p, b1, b2)


def kernel(features, edge_row, edge_col, edge_val, W1, b1, W2, b2):
    n, d = features.shape
    e = edge_row.shape[0]
    half = d // 2

    gran = NS * K * 8
    e_pad = -(-e // gran) * gran
    pad = e_pad - e
    ch = e_pad // (NS * K)
    col_p = jnp.pad(edge_col, (0, pad))
    row_p = jnp.pad(edge_row, (0, pad))
    val_p = jnp.pad(edge_val, (0, pad))
    val_bits = jax.lax.bitcast_convert_type(val_p, jnp.int32)
    col2 = jnp.stack([col_p, col_p + n])
    row2 = jnp.broadcast_to(row_p, (NC, e_pad))
    vb2 = jnp.broadcast_to(val_bits, (NC, e_pad))
    edata = jnp.stack([col2, row2, vb2], axis=1)
    edata = edata.reshape(NC, 3, NS, ch, K).transpose(0, 2, 3, 1, 4)
    table = features.reshape(n, NC, half).transpose(1, 0, 2)
    table = table.reshape(NC * n, half).astype(jnp.bfloat16)
    table = table.reshape(NC * n, half // (2 * L), 2, L)
    table = table.transpose(0, 1, 3, 2).reshape(NC * n, half)

    n_acc = -(-n // (NS * K)) * NS * K

    o1, o2 = _make_sc_spmm(n_acc, half, ch)(table, edata)

    perm = np.arange(d).reshape(d // (2 * L), 2, L).transpose(0, 2, 1)
    perm = perm.reshape(d)
    w2p = W2.T[perm]

    return _tc_tail(o1, o2, features, W1.T, w2p,
                    b1.reshape(1, d), b2.reshape(1, d), n, d, half)
